# baseline (speedup 1.0000x reference)
import jax
import jax.numpy as jnp
import numpy as np
from jax import lax
from jax.experimental import pallas as pl
from jax.experimental.pallas import tpu as pltpu
from jax.experimental.pallas import tpu_sc as plsc

S = 1024
N = 50
NMAX = 64
D = 128
K = 1024
EPS = 0.1
E_FAC = float(1.0 / np.sqrt(1.0 + EPS ** 2))
C_NOISE = float(E_FAC * EPS / np.sqrt(D))
T = 3 * N + 2
W = 2 * NMAX + D
NC, NS = 2, 16
NW = NC * NS
SPW = S // NW
TPAD = 160
LANES = 16
TROWS = 2 * K


def _sc_body(table, gidx, n1, n2, pe, out,
             idx_v, r0, r1, r2, n1_0, n1_1, n1_2, n2_0, n2_1, n2_2,
             zeros_s, tab_s,
             sg0, sg1, sg2, sn0, sn1, sn2, so0, so1, so2):
    wid = lax.axis_index("s") * NC + lax.axis_index("c")
    sid = lax.axis_index("s")
    base = wid * SPW

    pltpu.sync_copy(table.at[pl.ds(sid * (TROWS // NS), TROWS // NS)],
                    r0.at[pl.ds(0, TROWS // NS)])
    pltpu.sync_copy(r0.at[pl.ds(0, TROWS // NS)],
                    tab_s.at[pl.ds(sid * (TROWS // NS), TROWS // NS)])

    @pl.when(sid == 0)
    def _():
        def zero_row(t, c):
            for k in range(D // LANES):
                r1[t, pl.ds(LANES * k, LANES)] = jnp.zeros((LANES,), jnp.float32)
            return c

        lax.fori_loop(0, T, zero_row, 0)
        pltpu.sync_copy(r1, zeros_s)

    plsc.subcore_barrier()

    pltpu.sync_copy(gidx.at[pl.ds(base * TPAD, SPW * TPAD)], idx_v)

    def in_copies(s, rows, nn1, nn2, sg, sn):
        return (
            pltpu.make_async_copy(
                tab_s.at[idx_v.at[pl.ds(s * TPAD, 128)]],
                rows.at[pl.ds(0, 128)], sg),
            pltpu.make_async_copy(
                tab_s.at[idx_v.at[pl.ds(s * TPAD + 128, T - 128)]],
                rows.at[pl.ds(128, T - 128)], sg),
            pltpu.make_async_copy(n1.at[base + s], nn1, sn),
            pltpu.make_async_copy(n2.at[base + s], nn2, sn),
        )

    def issue_in(s, bufs):
        for c in in_copies(s, *bufs[:3], *bufs[3:5]):
            c.start()
        so = bufs[5]
        first = (wid == 0) & (s == 0)

        @pl.when(first)
        def _():
            pltpu.make_async_copy(pe, out.at[base + s, :, pl.ds(0, D)], so).start()

        @pl.when(jnp.logical_not(first))
        def _():
            pltpu.make_async_copy(
                zeros_s, out.at[base + s, :, pl.ds(0, D)], so).start()

    def out_copies(s, rows, so):
        return (
            pltpu.make_async_copy(
                rows, out.at[base + s, :, pl.ds(D, D)], so),
            pltpu.make_async_copy(
                zeros_s, out.at[base + s, :, pl.ds(0, D)], so),
        )

    def wait_out(s, bufs):
        for c in out_copies(s, bufs[0], bufs[5]):
            c.wait()

    def process(s, bufs):
        rows, nn1, nn2, sg, sn, so = bufs
        for c in in_copies(s, rows, nn1, nn2, sg, sn):
            c.wait()

        @plsc.parallel_loop(0, N + 1, unroll=8)
        def _(i):
            t0 = 3 * i
            for k in range(D // LANES):
                sl = pl.ds(LANES * k, LANES)
                rows[t0, sl] = rows[t0, sl] + C_NOISE * nn1[i, sl]
                rows[t0 + 1, sl] = rows[t0 + 1, sl] + C_NOISE * nn2[i, sl]

        pltpu.make_async_copy(rows, out.at[base + s, :, pl.ds(D, D)], so).start()

    c0 = (r0, n1_0, n2_0, sg0, sn0, so0)
    c1 = (r1, n1_1, n2_1, sg1, sn1, so1)
    c2 = (r2, n1_2, n2_2, sg2, sn2, so2)

    issue_in(0, c0)
    issue_in(1, c1)

    def body(j, c):
        q = 3 * j

        @pl.when(j > 0)
        def _():
            wait_out(q - 1, c2)

        issue_in(q + 2, c2)
        process(q, c0)
        wait_out(q, c0)
        issue_in(q + 3, c0)
        process(q + 1, c1)
        wait_out(q + 1, c1)
        issue_in(q + 4, c1)
        process(q + 2, c2)
        return c

    lax.fori_loop(0, (SPW - 2) // 3, body, 0)
    wait_out(SPW - 3, c2)
    process(SPW - 2, c0)
    process(SPW - 1, c1)
    wait_out(SPW - 2, c0)
    wait_out(SPW - 1, c1)


_sc_call = pl.kernel(
    _sc_body,
    out_type=jax.ShapeDtypeStruct((S, T, W), jnp.float32),
    mesh=plsc.VectorSubcoreMesh(
        core_axis_name="c", subcore_axis_name="s",
        num_cores=NC, num_subcores=NS),
    scratch_types=[
        pltpu.VMEM((SPW * TPAD,), jnp.int32),
        pltpu.VMEM((T, D), jnp.float32),
        pltpu.VMEM((T, D), jnp.float32),
        pltpu.VMEM((T, D), jnp.float32),
        pltpu.VMEM((N + 1, D), jnp.float32),
        pltpu.VMEM((N + 1, D), jnp.float32),
        pltpu.VMEM((N + 1, D), jnp.float32),
        pltpu.VMEM((N + 1, D), jnp.float32),
        pltpu.VMEM((N + 1, D), jnp.float32),
        pltpu.VMEM((N + 1, D), jnp.float32),
        pltpu.VMEM_SHARED((T, D), jnp.float32),
        pltpu.VMEM_SHARED((TROWS, D), jnp.float32),
        pltpu.SemaphoreType.DMA,
        pltpu.SemaphoreType.DMA,
        pltpu.SemaphoreType.DMA,
        pltpu.SemaphoreType.DMA,
        pltpu.SemaphoreType.DMA,
        pltpu.SemaphoreType.DMA,
        pltpu.SemaphoreType.DMA,
        pltpu.SemaphoreType.DMA,
        pltpu.SemaphoreType.DMA,
    ],
)


def kernel(example, label, shifts, noise1, noise2, mus_label, mus_class,
           positional_embedding):
    example = example.astype(jnp.int32)
    label = label.astype(jnp.int32)
    e0 = example[:, 0::2]
    e1 = example[:, 1::2]
    lab = label[:, :N] + K
    trip = jnp.stack([e0[:, :N], e1[:, :N], lab], axis=2).reshape(S, 3 * N)
    gidx = jnp.concatenate(
        [trip, e0[:, N:], e1[:, N:], jnp.zeros((S, TPAD - T), jnp.int32)],
        axis=1).reshape(S * TPAD)
    table = jnp.concatenate([E_FAC * mus_class, mus_label], axis=0)
    pe_slice = lax.dynamic_slice(
        positional_embedding[0], (shifts[0], 0), (T, 2 * NMAX))
    return _sc_call(table, gidx, noise1, noise2, pe_slice)


# speedup vs baseline: 1.0039x; 1.0039x over previous
import jax
import jax.numpy as jnp
import numpy as np
from jax import lax
from jax.experimental import pallas as pl
from jax.experimental.pallas import tpu as pltpu
from jax.experimental.pallas import tpu_sc as plsc

S = 1024
N = 50
NMAX = 64
D = 128
K = 1024
EPS = 0.1
E_FAC = float(1.0 / np.sqrt(1.0 + EPS ** 2))
C_NOISE = float(E_FAC * EPS / np.sqrt(D))
T = 3 * N + 2
W = 2 * NMAX + D
NC, NS = 2, 16
NW = NC * NS
SPW = S // NW
TPAD = 160
LANES = 16
TROWS = 2 * K


def _sc_body(table, gidx, n1, n2, pe, out,
             idx_v, r0, r1, r2, n1_0, n1_1, n1_2, n2_0, n2_1, n2_2,
             zeros_s, tab_s,
             sg0, sg1, sg2, sn0, sn1, sn2, so0, so1, so2):
    wid = lax.axis_index("s") * NC + lax.axis_index("c")
    sid = lax.axis_index("s")
    base = wid * SPW

    pltpu.sync_copy(table.at[pl.ds(sid * (TROWS // NS), TROWS // NS)],
                    r0.at[pl.ds(0, TROWS // NS)])
    pltpu.sync_copy(r0.at[pl.ds(0, TROWS // NS)],
                    tab_s.at[pl.ds(sid * (TROWS // NS), TROWS // NS)])

    @pl.when(sid == 0)
    def _():
        def zero_row(t, c):
            for k in range(D // LANES):
                r1[t, pl.ds(LANES * k, LANES)] = jnp.zeros((LANES,), jnp.float32)
            return c

        lax.fori_loop(0, T, zero_row, 0)
        pltpu.sync_copy(r1, zeros_s)

    plsc.subcore_barrier()

    pltpu.sync_copy(gidx.at[pl.ds(base * TPAD, SPW * TPAD)], idx_v)

    def in_copies(s, rows, nn1, nn2, sg, sn):
        return (
            pltpu.make_async_copy(
                tab_s.at[idx_v.at[pl.ds(s * TPAD, 128)]],
                rows.at[pl.ds(0, 128)], sg),
            pltpu.make_async_copy(
                tab_s.at[idx_v.at[pl.ds(s * TPAD + 128, T - 128)]],
                rows.at[pl.ds(128, T - 128)], sg),
            pltpu.make_async_copy(n1.at[base + s], nn1, sn),
            pltpu.make_async_copy(n2.at[base + s], nn2, sn),
        )

    def issue_in(s, bufs):
        for c in in_copies(s, *bufs[:3], *bufs[3:5]):
            c.start()
        so = bufs[5]
        first = (wid == 0) & (s == 0)

        @pl.when(first)
        def _():
            pltpu.make_async_copy(pe, out.at[base + s, :, pl.ds(0, D)], so).start()

        @pl.when(jnp.logical_not(first))
        def _():
            pltpu.make_async_copy(
                zeros_s, out.at[base + s, :, pl.ds(0, D)], so).start()

    def out_copies(s, rows, so):
        return (
            pltpu.make_async_copy(
                rows, out.at[base + s, :, pl.ds(D, D)], so),
            pltpu.make_async_copy(
                zeros_s, out.at[base + s, :, pl.ds(0, D)], so),
        )

    def wait_out(s, bufs):
        for c in out_copies(s, bufs[0], bufs[5]):
            c.wait()

    def process(s, bufs):
        rows, nn1, nn2, sg, sn, so = bufs
        for c in in_copies(s, rows, nn1, nn2, sg, sn):
            c.wait()

        @plsc.parallel_loop(0, N + 1, unroll=4)
        def _(i):
            t0 = 3 * i
            for k in range(D // LANES):
                sl = pl.ds(LANES * k, LANES)
                rows[t0, sl] = rows[t0, sl] + C_NOISE * nn1[i, sl]
                rows[t0 + 1, sl] = rows[t0 + 1, sl] + C_NOISE * nn2[i, sl]

        pltpu.make_async_copy(rows, out.at[base + s, :, pl.ds(D, D)], so).start()

    c0 = (r0, n1_0, n2_0, sg0, sn0, so0)
    c1 = (r1, n1_1, n2_1, sg1, sn1, so1)
    c2 = (r2, n1_2, n2_2, sg2, sn2, so2)

    issue_in(0, c0)
    issue_in(1, c1)

    def body(j, c):
        q = 3 * j

        @pl.when(j > 0)
        def _():
            wait_out(q - 1, c2)

        issue_in(q + 2, c2)
        process(q, c0)
        wait_out(q, c0)
        issue_in(q + 3, c0)
        process(q + 1, c1)
        wait_out(q + 1, c1)
        issue_in(q + 4, c1)
        process(q + 2, c2)
        return c

    lax.fori_loop(0, (SPW - 2) // 3, body, 0)
    wait_out(SPW - 3, c2)
    process(SPW - 2, c0)
    process(SPW - 1, c1)
    wait_out(SPW - 2, c0)
    wait_out(SPW - 1, c1)


_sc_call = pl.kernel(
    _sc_body,
    out_type=jax.ShapeDtypeStruct((S, T, W), jnp.float32),
    mesh=plsc.VectorSubcoreMesh(
        core_axis_name="c", subcore_axis_name="s",
        num_cores=NC, num_subcores=NS),
    scratch_types=[
        pltpu.VMEM((SPW * TPAD,), jnp.int32),
        pltpu.VMEM((T, D), jnp.float32),
        pltpu.VMEM((T, D), jnp.float32),
        pltpu.VMEM((T, D), jnp.float32),
        pltpu.VMEM((N + 1, D), jnp.float32),
        pltpu.VMEM((N + 1, D), jnp.float32),
        pltpu.VMEM((N + 1, D), jnp.float32),
        pltpu.VMEM((N + 1, D), jnp.float32),
        pltpu.VMEM((N + 1, D), jnp.float32),
        pltpu.VMEM((N + 1, D), jnp.float32),
        pltpu.VMEM_SHARED((T, D), jnp.float32),
        pltpu.VMEM_SHARED((TROWS, D), jnp.float32),
        pltpu.SemaphoreType.DMA,
        pltpu.SemaphoreType.DMA,
        pltpu.SemaphoreType.DMA,
        pltpu.SemaphoreType.DMA,
        pltpu.SemaphoreType.DMA,
        pltpu.SemaphoreType.DMA,
        pltpu.SemaphoreType.DMA,
        pltpu.SemaphoreType.DMA,
        pltpu.SemaphoreType.DMA,
    ],
)


def kernel(example, label, shifts, noise1, noise2, mus_label, mus_class,
           positional_embedding):
    example = example.astype(jnp.int32)
    label = label.astype(jnp.int32)
    e0 = example[:, 0::2]
    e1 = example[:, 1::2]
    lab = label[:, :N] + K
    trip = jnp.stack([e0[:, :N], e1[:, :N], lab], axis=2).reshape(S, 3 * N)
    gidx = jnp.concatenate(
        [trip, e0[:, N:], e1[:, N:], jnp.zeros((S, TPAD - T), jnp.int32)],
        axis=1).reshape(S * TPAD)
    table = jnp.concatenate([E_FAC * mus_class, mus_label], axis=0)
    pe_slice = lax.dynamic_slice(
        positional_embedding[0], (shifts[0], 0), (T, 2 * NMAX))
    return _sc_call(table, gidx, noise1, noise2, pe_slice)


# speedup vs baseline: 1.0181x; 1.0142x over previous
import jax
import jax.numpy as jnp
import numpy as np
from jax import lax
from jax.experimental import pallas as pl
from jax.experimental.pallas import tpu as pltpu
from jax.experimental.pallas import tpu_sc as plsc

S = 1024
N = 50
NMAX = 64
D = 128
K = 1024
EPS = 0.1
E_FAC = float(1.0 / np.sqrt(1.0 + EPS ** 2))
C_NOISE = float(E_FAC * EPS / np.sqrt(D))
T = 3 * N + 2
W = 2 * NMAX + D
NC, NS = 2, 16
NW = NC * NS
SPW = S // NW
TPAD = 160
LANES = 16
TROWS = 2 * K


def _sc_body(table, gidx, n1, n2, pe, out,
             idx_v, r0, r1, r2, n1_0, n1_1, n1_2, n2_0, n2_1, n2_2,
             zeros_s, tab_s,
             sg0, sg1, sg2, sn0, sn1, sn2, so0, so1, so2):
    wid = lax.axis_index("s") * NC + lax.axis_index("c")
    sid = lax.axis_index("s")
    base = wid * SPW

    pltpu.sync_copy(table.at[pl.ds(sid * (TROWS // NS), TROWS // NS)],
                    r0.at[pl.ds(0, TROWS // NS)])
    pltpu.sync_copy(r0.at[pl.ds(0, TROWS // NS)],
                    tab_s.at[pl.ds(sid * (TROWS // NS), TROWS // NS)])

    @pl.when(sid == 0)
    def _():
        def zero_row(t, c):
            for k in range(D // LANES):
                r1[t, pl.ds(LANES * k, LANES)] = jnp.zeros((LANES,), jnp.float32)
            return c

        lax.fori_loop(0, T, zero_row, 0)
        pltpu.sync_copy(r1, zeros_s)

    plsc.subcore_barrier()

    pltpu.sync_copy(gidx.at[pl.ds(base * TPAD, SPW * TPAD)], idx_v)

    def in_copies(s, rows, nn1, nn2, sg, sn):
        return (
            pltpu.make_async_copy(
                tab_s.at[idx_v.at[pl.ds(s * TPAD, 128)]],
                rows.at[pl.ds(0, 128)], sg),
            pltpu.make_async_copy(
                tab_s.at[idx_v.at[pl.ds(s * TPAD + 128, T - 128)]],
                rows.at[pl.ds(128, T - 128)], sg),
            pltpu.make_async_copy(n1.at[base + s], nn1, sn),
            pltpu.make_async_copy(n2.at[base + s], nn2, sn),
        )

    def issue_in(s, bufs):
        for c in in_copies(s, *bufs[:3], *bufs[3:5]):
            c.start()
        so = bufs[5]
        first = (wid == 0) & (s == 0)

        @pl.when(first)
        def _():
            pltpu.make_async_copy(pe, out.at[base + s, :, pl.ds(0, D)], so).start()

        @pl.when(jnp.logical_not(first))
        def _():
            pltpu.make_async_copy(
                zeros_s, out.at[base + s, :, pl.ds(0, D)], so).start()

    def out_copies(s, rows, so):
        return (
            pltpu.make_async_copy(
                rows, out.at[base + s, :, pl.ds(D, D)], so),
            pltpu.make_async_copy(
                zeros_s, out.at[base + s, :, pl.ds(0, D)], so),
        )

    def wait_out(s, bufs):
        for c in out_copies(s, bufs[0], bufs[5]):
            c.wait()

    def process(s, bufs):
        rows, nn1, nn2, sg, sn, so = bufs
        for c in in_copies(s, rows, nn1, nn2, sg, sn):
            c.wait()

        @plsc.parallel_loop(0, N + 1, unroll=4)
        def _(i):
            t0 = 3 * i
            for k in range(D // LANES):
                sl = pl.ds(LANES * k, LANES)
                rows[t0, sl] = rows[t0, sl] + C_NOISE * nn1[i, sl]
                rows[t0 + 1, sl] = rows[t0 + 1, sl] + C_NOISE * nn2[i, sl]

        pltpu.make_async_copy(rows, out.at[base + s, :, pl.ds(D, D)], so).start()

    c0 = (r0, n1_0, n2_0, sg0, sn0, so0)
    c1 = (r1, n1_1, n2_1, sg1, sn1, so1)
    c2 = (r2, n1_2, n2_2, sg2, sn2, so2)

    issue_in(0, c0)
    issue_in(1, c1)

    def body(j, c):
        q = 3 * j
        process(q, c0)

        @pl.when(j > 0)
        def _():
            wait_out(q - 1, c2)

        issue_in(q + 2, c2)
        process(q + 1, c1)
        wait_out(q, c0)
        issue_in(q + 3, c0)
        process(q + 2, c2)
        wait_out(q + 1, c1)
        issue_in(q + 4, c1)
        return c

    lax.fori_loop(0, (SPW - 2) // 3, body, 0)
    wait_out(SPW - 3, c2)
    process(SPW - 2, c0)
    process(SPW - 1, c1)
    wait_out(SPW - 2, c0)
    wait_out(SPW - 1, c1)


_sc_call = pl.kernel(
    _sc_body,
    out_type=jax.ShapeDtypeStruct((S, T, W), jnp.float32),
    mesh=plsc.VectorSubcoreMesh(
        core_axis_name="c", subcore_axis_name="s",
        num_cores=NC, num_subcores=NS),
    scratch_types=[
        pltpu.VMEM((SPW * TPAD,), jnp.int32),
        pltpu.VMEM((T, D), jnp.float32),
        pltpu.VMEM((T, D), jnp.float32),
        pltpu.VMEM((T, D), jnp.float32),
        pltpu.VMEM((N + 1, D), jnp.float32),
        pltpu.VMEM((N + 1, D), jnp.float32),
        pltpu.VMEM((N + 1, D), jnp.float32),
        pltpu.VMEM((N + 1, D), jnp.float32),
        pltpu.VMEM((N + 1, D), jnp.float32),
        pltpu.VMEM((N + 1, D), jnp.float32),
        pltpu.VMEM_SHARED((T, D), jnp.float32),
        pltpu.VMEM_SHARED((TROWS, D), jnp.float32),
        pltpu.SemaphoreType.DMA,
        pltpu.SemaphoreType.DMA,
        pltpu.SemaphoreType.DMA,
        pltpu.SemaphoreType.DMA,
        pltpu.SemaphoreType.DMA,
        pltpu.SemaphoreType.DMA,
        pltpu.SemaphoreType.DMA,
        pltpu.SemaphoreType.DMA,
        pltpu.SemaphoreType.DMA,
    ],
)


def kernel(example, label, shifts, noise1, noise2, mus_label, mus_class,
           positional_embedding):
    example = example.astype(jnp.int32)
    label = label.astype(jnp.int32)
    e0 = example[:, 0::2]
    e1 = example[:, 1::2]
    lab = label[:, :N] + K
    trip = jnp.stack([e0[:, :N], e1[:, :N], lab], axis=2).reshape(S, 3 * N)
    gidx = jnp.concatenate(
        [trip, e0[:, N:], e1[:, N:], jnp.zeros((S, TPAD - T), jnp.int32)],
        axis=1).reshape(S * TPAD)
    table = jnp.concatenate([E_FAC * mus_class, mus_label], axis=0)
    pe_slice = lax.dynamic_slice(
        positional_embedding[0], (shifts[0], 0), (T, 2 * NMAX))
    return _sc_call(table, gidx, noise1, noise2, pe_slice)
